# baseline (speedup 1.0000x reference)
import functools

import jax
import jax.numpy as jnp
from jax import lax
from jax.experimental import pallas as pl
from jax.experimental.pallas import tpu as pltpu
from jax.experimental.pallas import tpu_sc as plsc

M = 100000
N = 50000
D = 128

NC = 2
NS = 16
NW = NC * NS

CHUNK = 112
NCHUNK = 14
C = CHUNK * NCHUNK
NBUF = 6


@functools.cache
def _get_sc_scatter():
    mesh = plsc.VectorSubcoreMesh(
        core_axis_name="c", subcore_axis_name="s",
        num_cores=NC, num_subcores=NS)

    @functools.partial(
        pl.kernel,
        mesh=mesh,
        out_type=(),
        scratch_types=[
            pltpu.VMEM((NCHUNK, CHUNK), jnp.int32),
            pltpu.VMEM((NBUF, CHUNK, D), jnp.float32),
            pltpu.SemaphoreType.DMA,
            [pltpu.SemaphoreType.DMA] * NBUF,
            [pltpu.SemaphoreType.DMA] * NBUF,
        ],
    )
    def _sc_scatter(xl_hbm, idx_hbm, out_hbm, idx_v, rows_v, isem, gsems,
                    ssems):
        c = lax.axis_index("c")
        s = lax.axis_index("s")
        wid = s * NC + c
        start = jnp.minimum(wid * C, N - C)

        icopies = [
            pltpu.async_copy(
                idx_hbm.at[pl.ds(start + j * CHUNK, CHUNK)], idx_v.at[j],
                isem)
            for j in range(NCHUNK)
        ]

        def gather(j):
            b = j % NBUF
            return pltpu.async_copy(
                xl_hbm.at[pl.ds(start + j * CHUNK, CHUNK), :],
                rows_v.at[b], gsems[b])

        gathers = [gather(0), gather(1), gather(2)]
        for icopy in icopies:
            icopy.wait()
        scats = [None] * NCHUNK
        for j in range(NCHUNK):
            b = j % NBUF
            prev = j + 3 - NBUF
            if prev >= 0:
                scats[prev].wait()
            if j + 3 < NCHUNK:
                gathers.append(gather(j + 3))
            gathers[j].wait()
            scats[j] = pltpu.async_copy(
                rows_v.at[b], out_hbm.at[idx_v.at[j]], ssems[b])
        for j in range(max(0, NCHUNK - (NBUF - 3)), NCHUNK):
            scats[j].wait()

    return _sc_scatter


_ZBLK = 5000


def _zero_body(o_ref):
    o_ref[...] = jnp.zeros_like(o_ref)


_zero_fill = pl.pallas_call(
    _zero_body,
    out_shape=jax.ShapeDtypeStruct((M, D), jnp.float32),
    grid=(M // _ZBLK,),
    out_specs=pl.BlockSpec((_ZBLK, D), lambda i: (i, 0)),
)


def kernel(xl, idxs, up_shape):
    del up_shape
    idxs = idxs.astype(jnp.int32)
    out_ref = jax.new_ref(_zero_fill())
    _get_sc_scatter()(xl, idxs, out_ref)
    return jax.freeze(out_ref)


# speedup vs baseline: 1.0077x; 1.0077x over previous
import functools

import jax
import jax.numpy as jnp
from jax import lax
from jax.experimental import pallas as pl
from jax.experimental.pallas import tpu as pltpu
from jax.experimental.pallas import tpu_sc as plsc

M = 100000
N = 50000
D = 128

NC = 2
NS = 16
NW = NC * NS

CHUNK = 112
NCHUNK = 14
C = CHUNK * NCHUNK
NBUF = 6


@functools.cache
def _get_sc_scatter():
    mesh = plsc.VectorSubcoreMesh(
        core_axis_name="c", subcore_axis_name="s",
        num_cores=NC, num_subcores=NS)

    @functools.partial(
        pl.kernel,
        mesh=mesh,
        out_type=(),
        scratch_types=[
            pltpu.VMEM((NCHUNK, CHUNK), jnp.int32),
            pltpu.VMEM((NBUF, CHUNK, D), jnp.float32),
            pltpu.SemaphoreType.DMA,
            [pltpu.SemaphoreType.DMA] * NBUF,
            [pltpu.SemaphoreType.DMA] * NBUF,
        ],
    )
    def _sc_scatter(xl_hbm, idx_hbm, out_hbm, idx_v, rows_v, isem, gsems,
                    ssems):
        c = lax.axis_index("c")
        s = lax.axis_index("s")
        wid = s * NC + c
        start = jnp.minimum(wid * C, N - C)

        icopies = [
            pltpu.async_copy(
                idx_hbm.at[pl.ds(start + j * CHUNK, CHUNK)], idx_v.at[j],
                isem)
            for j in range(NCHUNK)
        ]

        def gather(j):
            b = j % NBUF
            return pltpu.async_copy(
                xl_hbm.at[pl.ds(start + j * CHUNK, CHUNK), :],
                rows_v.at[b], gsems[b])

        gathers = [gather(0), gather(1)]
        for icopy in icopies:
            icopy.wait()
        scats = [None] * NCHUNK
        for j in range(NCHUNK):
            b = j % NBUF
            prev = j + 2 - NBUF
            if prev >= 0:
                scats[prev].wait()
            if j + 2 < NCHUNK:
                gathers.append(gather(j + 2))
            gathers[j].wait()
            scats[j] = pltpu.async_copy(
                rows_v.at[b], out_hbm.at[idx_v.at[j]], ssems[b])
        for j in range(max(0, NCHUNK - (NBUF - 2)), NCHUNK):
            scats[j].wait()

    return _sc_scatter


_ZBLK = 10000


def _zero_body(o_ref):
    o_ref[...] = jnp.zeros_like(o_ref)


_zero_fill = pl.pallas_call(
    _zero_body,
    out_shape=jax.ShapeDtypeStruct((M, D), jnp.float32),
    grid=(M // _ZBLK,),
    out_specs=pl.BlockSpec((_ZBLK, D), lambda i: (i, 0)),
)


def kernel(xl, idxs, up_shape):
    del up_shape
    idxs = idxs.astype(jnp.int32)
    out_ref = jax.new_ref(_zero_fill())
    _get_sc_scatter()(xl, idxs, out_ref)
    return jax.freeze(out_ref)
